# baseline (speedup 1.0000x reference)
import functools

import jax
import jax.numpy as jnp
from jax import lax
from jax.experimental import pallas as pl
from jax.experimental.pallas import tpu as pltpu
from jax.experimental.pallas import tpu_sc as plsc

N_IDS = 425984
NC = 2
NS = 16
NW = NC * NS
B_PER_W = N_IDS // NW

_mesh = plsc.VectorSubcoreMesh(core_axis_name="c", subcore_axis_name="s")


@functools.partial(
    pl.kernel,
    mesh=_mesh,
    out_type=jax.ShapeDtypeStruct((N_IDS,), jnp.float32),
    scratch_types=[
        pltpu.VMEM((B_PER_W,), jnp.int32),
        pltpu.VMEM((B_PER_W,), jnp.float32),
        pltpu.SemaphoreType.DMA,
    ],
)
def _gather_kernel(ids_hbm, table_hbm, out_hbm, idx_v, rows_v, sem):
    wid = lax.axis_index("s") * NC + lax.axis_index("c")
    base = wid * B_PER_W
    pltpu.sync_copy(ids_hbm.at[pl.ds(base, B_PER_W)], idx_v)
    pltpu.async_copy(table_hbm.at[idx_v], rows_v, sem).wait()
    pltpu.sync_copy(rows_v, out_hbm.at[pl.ds(base, B_PER_W)])


def kernel(ids, kernel):
    gathered = _gather_kernel(ids.astype(jnp.int32), kernel)
    return gathered.reshape(N_IDS, 1)


# speedup vs baseline: 1.1456x; 1.1456x over previous
import functools

import jax
import jax.numpy as jnp
from jax import lax
from jax.experimental import pallas as pl
from jax.experimental.pallas import tpu as pltpu
from jax.experimental.pallas import tpu_sc as plsc

N_IDS = 425984
NUM_BUCKETS = 1000000
TBL_PAD = 1048576
NC = 2
NS = 16
NW = NC * NS
B_PER_W = N_IDS // NW
SEG = TBL_PAD // NS

_mesh = plsc.VectorSubcoreMesh(core_axis_name="c", subcore_axis_name="s")


@functools.partial(
    pl.kernel,
    mesh=_mesh,
    out_type=jax.ShapeDtypeStruct((N_IDS,), jnp.float32),
    scratch_types=[
        pltpu.VMEM((B_PER_W,), jnp.int32),
        pltpu.VMEM((B_PER_W,), jnp.float32),
        pltpu.VMEM_SHARED((TBL_PAD,), jnp.float32),
        pltpu.SemaphoreType.DMA,
    ],
)
def _gather_kernel(ids_hbm, table_hbm, out_hbm, idx_v, rows_v, tbl_sh, sem):
    cid = lax.axis_index("c")
    sid = lax.axis_index("s")
    wid = sid * NC + cid
    base = wid * B_PER_W
    pltpu.sync_copy(ids_hbm.at[pl.ds(base, B_PER_W)], idx_v)
    pltpu.sync_copy(table_hbm.at[pl.ds(sid * SEG, SEG)],
                    tbl_sh.at[pl.ds(sid * SEG, SEG)])
    plsc.subcore_barrier()
    pltpu.async_copy(tbl_sh.at[idx_v], rows_v, sem).wait()
    pltpu.sync_copy(rows_v, out_hbm.at[pl.ds(base, B_PER_W)])


def kernel(ids, kernel):
    table = jnp.concatenate(
        [kernel, jnp.zeros((TBL_PAD - NUM_BUCKETS,), jnp.float32)])
    gathered = _gather_kernel(ids.astype(jnp.int32), table)
    return gathered.reshape(N_IDS, 1)


# speedup vs baseline: 1.2495x; 1.0907x over previous
import functools

import jax
import jax.numpy as jnp
from jax import lax
from jax.experimental import pallas as pl
from jax.experimental.pallas import tpu as pltpu
from jax.experimental.pallas import tpu_sc as plsc

N_IDS = 425984
NUM_BUCKETS = 1000000
NC = 2
NS = 16
NW = NC * NS
B_PER_W = N_IDS // NW
SEG = 62464
SEG_LAST = 999936 - (NS - 1) * SEG
TAIL_OFF = 999936
SH_SIZE = TAIL_OFF + 512

_mesh = plsc.VectorSubcoreMesh(core_axis_name="c", subcore_axis_name="s")


@functools.partial(
    pl.kernel,
    mesh=_mesh,
    out_type=jax.ShapeDtypeStruct((N_IDS,), jnp.float32),
    scratch_types=[
        pltpu.VMEM((B_PER_W,), jnp.int32),
        pltpu.VMEM((B_PER_W,), jnp.float32),
        pltpu.VMEM((512,), jnp.float32),
        pltpu.VMEM_SHARED((SH_SIZE,), jnp.float32),
        pltpu.SemaphoreType.DMA,
        pltpu.SemaphoreType.DMA,
    ],
)
def _gather_kernel(ids_hbm, table_hbm, out_hbm, idx_v, rows_v, tail_v,
                   tbl_sh, g_sem, t_sem):
    cid = lax.axis_index("c")
    sid = lax.axis_index("s")
    wid = sid * NC + cid
    base = wid * B_PER_W

    @pl.when(sid < NS - 1)
    def _stage_main():
        pltpu.async_copy(table_hbm.at[pl.ds(sid * SEG, SEG)],
                         tbl_sh.at[pl.ds(sid * SEG, SEG)], t_sem)

    @pl.when(sid == NS - 1)
    def _stage_last():
        pltpu.async_copy(table_hbm.at[pl.ds((NS - 1) * SEG, SEG_LAST)],
                         tbl_sh.at[pl.ds((NS - 1) * SEG, SEG_LAST)], t_sem)

    @pl.when(sid == 0)
    def _stage_tail():
        pltpu.sync_copy(table_hbm.at[pl.ds(TAIL_OFF, 64)],
                        tail_v.at[pl.ds(0, 64)])
        pltpu.sync_copy(tail_v, tbl_sh.at[pl.ds(TAIL_OFF, 512)])

    pltpu.sync_copy(ids_hbm.at[pl.ds(base, B_PER_W)], idx_v)

    @pl.when(sid < NS - 1)
    def _wait_main():
        pltpu.make_async_copy(table_hbm.at[pl.ds(sid * SEG, SEG)],
                              tbl_sh.at[pl.ds(sid * SEG, SEG)], t_sem).wait()

    @pl.when(sid == NS - 1)
    def _wait_last():
        pltpu.make_async_copy(
            table_hbm.at[pl.ds((NS - 1) * SEG, SEG_LAST)],
            tbl_sh.at[pl.ds((NS - 1) * SEG, SEG_LAST)], t_sem).wait()

    plsc.subcore_barrier()
    pltpu.async_copy(tbl_sh.at[idx_v], rows_v, g_sem).wait()
    pltpu.sync_copy(rows_v, out_hbm.at[pl.ds(base, B_PER_W)])


def kernel(ids, kernel):
    gathered = _gather_kernel(ids.astype(jnp.int32), kernel)
    return gathered.reshape(N_IDS, 1)
